# baseline (speedup 1.0000x reference)
import functools

import jax
import jax.numpy as jnp
from jax import lax
from jax.experimental import pallas as pl
from jax.experimental.pallas import tpu as pltpu
from jax.experimental.pallas import tpu_sc as plsc

B, S, D = 4096, 200, 64
N_TEST, N_TAG, N_TT, N_QN = 10000, 1000, 16, 64
NW = 32
TOK_PER_W = B * S // NW
L = 16
CH = 40
NCH = TOK_PER_W // CH
NST = 2
NT = 5
TAG0, TT0, QN0 = S, S + N_TAG, S + N_TAG + N_TT
NSMALL = S + N_TAG + N_TT + N_QN


def _body(tests, questions, tags, ttypes, qnums,
          w_test, w_quest, w_small,
          out, bufs, outbuf, smallv, idxb,
          sem_i, sem_g, sem_o):
    wid = lax.axis_index("s") * 2 + lax.axis_index("c")
    tok0 = wid * TOK_PER_W
    last = NCH - 1

    pltpu.sync_copy(w_small, smallv)

    idx_hbms = (questions, tests, tags, ttypes, qnums)
    w_refs = (w_quest, w_test)

    def fire_idx(c, slot):
        base = tok0 + c * CH
        for i in range(NT):
            pltpu.async_copy(idx_hbms[i].at[pl.ds(base, CH)],
                             idxb.at[i, slot], sem_i)

    def wait_idx():
        for i in range(NT):
            pltpu.make_async_copy(idx_hbms[i].at[pl.ds(0, CH)],
                                  idxb.at[i, 0], sem_i).wait()

    def fire_gather(slot, par):
        for i in range(NST):
            pltpu.async_copy(w_refs[i].at[idxb.at[i, slot]],
                             bufs.at[i, par], sem_g)

    def wait_gather():
        for i in range(NST):
            pltpu.make_async_copy(w_refs[i].at[idxb.at[i, 0]],
                                  bufs.at[i, 0], sem_g).wait()

    def wait_out():
        pltpu.make_async_copy(outbuf.at[0], out.at[pl.ds(0, CH)],
                              sem_o).wait()

    fire_idx(0, 0)
    fire_idx(1, 1)
    wait_idx()
    fire_gather(0, 0)

    lanes = lax.iota(jnp.int32, L)

    def step(cc, carry):
        for j in (0, 1, 2, 3):
            par = j % 2
            c = cc * 4 + j
            wait_gather()
            wait_idx()
            fire_gather((j + 1) % 4, 1 - par)
            fire_idx(jnp.minimum(c + 2, last), (j + 2) % 4)

            @pl.when(c >= 2)
            def _():
                wait_out()

            s0 = lax.rem(c, S // CH) * CH

            def tok_body(t, carry2):
                tsplat = jnp.broadcast_to(t, (L,))
                tag_a = (plsc.load_gather(idxb.at[2, j], [tsplat]) + TAG0) * D
                tt_a = (plsc.load_gather(idxb.at[3, j], [tsplat]) + TT0) * D
                qn_a = (plsc.load_gather(idxb.at[4, j], [tsplat]) + QN0) * D
                p0 = (s0 + t) * D
                for kp in range(D // (2 * L)):
                    qp = plsc.unpack(bufs[0, par, t, pl.ds(kp * 2 * L, 2 * L)],
                                     format=plsc.PackFormat.INTERLEAVED,
                                     preferred_element_type=jnp.float32)
                    tp = plsc.unpack(bufs[1, par, t, pl.ds(kp * 2 * L, 2 * L)],
                                     format=plsc.PackFormat.INTERLEAVED,
                                     preferred_element_type=jnp.float32)
                    for h in range(2):
                        k = kp * 2 + h
                        ds = pl.ds(k * L, L)
                        col = lanes + (k * L)
                        v = smallv[pl.ds(p0 + k * L, L)]
                        v = v + qp[h]
                        v = v + tp[h]
                        v = v + plsc.load_gather(smallv, [tag_a + col])
                        v = v + plsc.load_gather(smallv, [tt_a + col])
                        v = v + plsc.load_gather(smallv, [qn_a + col])
                        outbuf[par, t, ds] = v
                return carry2

            lax.fori_loop(0, CH, tok_body, 0, unroll=2)

            pltpu.async_copy(outbuf.at[par],
                             out.at[pl.ds(tok0 + c * CH, CH)], sem_o)
        return carry

    lax.fori_loop(0, NCH // 4, step, 0)

    wait_out()
    wait_out()
    wait_gather()
    wait_idx()


@jax.jit
def _run(tests, questions, tags, ttypes, qnums, w_test, w_quest, w_small):
    mesh = plsc.VectorSubcoreMesh(core_axis_name="c", subcore_axis_name="s")
    fn = pl.kernel(
        _body,
        out_type=jax.ShapeDtypeStruct((B * S, D), jnp.float32),
        mesh=mesh,
        compiler_params=pltpu.CompilerParams(use_tc_tiling_on_sc=False,
                                             needs_layout_passes=False),
        scratch_types=[
            pltpu.VMEM((NST, 2, CH, D), jnp.bfloat16),
            pltpu.VMEM((2, CH, D), jnp.float32),
            pltpu.VMEM((NSMALL * D,), jnp.float32),
            pltpu.VMEM((NT, 4, CH), jnp.int32),
            pltpu.SemaphoreType.DMA,
            pltpu.SemaphoreType.DMA,
            pltpu.SemaphoreType.DMA,
        ],
    )
    return fn(tests, questions, tags, ttypes, qnums, w_test, w_quest,
              w_small)


_PERM = tuple(
    32 * g + (16 * (i % 2)) + (i // 2)
    for g in range(2) for i in range(32))


def kernel(tests, questions, tags, test_types, question_num,
           W_test, W_question, W_tag, W_test_type, W_pos, W_qnum):
    flat = lambda x: x.reshape(-1).astype(jnp.int32)
    perm = jnp.array(_PERM, jnp.int32)
    w_small = jnp.concatenate(
        [W_pos, W_tag, W_test_type, W_qnum], axis=0).reshape(-1)
    out = _run(flat(tests), flat(questions), flat(tags), flat(test_types),
               flat(question_num),
               W_test[:, perm].astype(jnp.bfloat16),
               W_question[:, perm].astype(jnp.bfloat16), w_small)
    return out.reshape(B, S, D)


# speedup vs baseline: 1.0097x; 1.0097x over previous
import functools

import jax
import jax.numpy as jnp
from jax import lax
from jax.experimental import pallas as pl
from jax.experimental.pallas import tpu as pltpu
from jax.experimental.pallas import tpu_sc as plsc

B, S, D = 4096, 200, 64
N_TEST, N_TAG, N_TT, N_QN = 10000, 1000, 16, 64
NW = 32
TOK_PER_W = B * S // NW
L = 16
CH = 40
NCH = TOK_PER_W // CH
NST = 2
NT = 5
TAG0, TT0, QN0 = S, S + N_TAG, S + N_TAG + N_TT
NSMALL = S + N_TAG + N_TT + N_QN


def _body(tests, questions, tags, ttypes, qnums,
          w_test, w_quest, w_small,
          out, bufs, outbuf, smallv, idxb, packedv,
          sem_i, sem_g, sem_o):
    wid = lax.axis_index("s") * 2 + lax.axis_index("c")
    tok0 = wid * TOK_PER_W
    last = NCH - 1

    pltpu.sync_copy(w_small, smallv)

    idx_hbms = (questions, tests, tags, ttypes, qnums)
    w_refs = (w_quest, w_test)

    def fire_idx(c, slot):
        base = tok0 + c * CH
        for i in range(NT):
            pltpu.async_copy(idx_hbms[i].at[pl.ds(base, CH)],
                             idxb.at[i, slot, pl.ds(0, CH)], sem_i)

    def wait_idx():
        for i in range(NT):
            pltpu.make_async_copy(idx_hbms[i].at[pl.ds(0, CH)],
                                  idxb.at[i, 0, pl.ds(0, CH)], sem_i).wait()

    def fire_gather(slot, par):
        for i in range(NST):
            pltpu.async_copy(w_refs[i].at[idxb.at[i, slot, pl.ds(0, CH)]],
                             bufs.at[i, par], sem_g)

    def wait_gather():
        for i in range(NST):
            pltpu.make_async_copy(w_refs[i].at[idxb.at[i, 0, pl.ds(0, CH)]],
                                  bufs.at[i, 0], sem_g).wait()

    def wait_out():
        pltpu.make_async_copy(outbuf.at[0], out.at[pl.ds(0, CH)],
                              sem_o).wait()

    fire_idx(0, 0)
    fire_idx(1, 1)
    wait_idx()
    fire_gather(0, 0)

    lanes = lax.iota(jnp.int32, L)

    def step(cc, carry):
        for j in (0, 1, 2, 3):
            par = j % 2
            c = cc * 4 + j
            wait_gather()
            wait_idx()
            fire_gather((j + 1) % 4, 1 - par)
            fire_idx(jnp.minimum(c + 2, last), (j + 2) % 4)

            @pl.when(c >= 2)
            def _():
                wait_out()

            s0 = lax.rem(c, S // CH) * CH

            for gi in range(3):
                dsg = pl.ds(gi * L, L)
                packedv[dsg] = (idxb[2, j, dsg] * 1024
                                + idxb[3, j, dsg] * 64 + idxb[4, j, dsg])

            def tok_body(t, carry2):
                tsplat = jnp.broadcast_to(t, (L,))
                p = plsc.load_gather(packedv, [tsplat])
                tag_a = ((p >> 10) + TAG0) * D
                tt_a = (((p >> 6) & 15) + TT0) * D
                qn_a = ((p & 63) + QN0) * D
                p0 = (s0 + t) * D
                for k in range(D // L):
                    ds = pl.ds(k * L, L)
                    col = lanes + (k * L)
                    v = smallv[pl.ds(p0 + k * L, L)]
                    v = v + bufs[0, par, t, ds]
                    v = v + bufs[1, par, t, ds]
                    v = v + plsc.load_gather(smallv, [tag_a + col])
                    v = v + plsc.load_gather(smallv, [tt_a + col])
                    v = v + plsc.load_gather(smallv, [qn_a + col])
                    outbuf[par, t, ds] = v
                return carry2

            lax.fori_loop(0, CH, tok_body, 0, unroll=2)

            pltpu.async_copy(outbuf.at[par],
                             out.at[pl.ds(tok0 + c * CH, CH)], sem_o)
        return carry

    lax.fori_loop(0, NCH // 4, step, 0)

    wait_out()
    wait_out()
    wait_gather()
    wait_idx()


@jax.jit
def _run(tests, questions, tags, ttypes, qnums, w_test, w_quest, w_small):
    mesh = plsc.VectorSubcoreMesh(core_axis_name="c", subcore_axis_name="s")
    fn = pl.kernel(
        _body,
        out_type=jax.ShapeDtypeStruct((B * S, D), jnp.float32),
        mesh=mesh,
        compiler_params=pltpu.CompilerParams(use_tc_tiling_on_sc=False,
                                             needs_layout_passes=False),
        scratch_types=[
            pltpu.VMEM((NST, 2, CH, D), jnp.float32),
            pltpu.VMEM((2, CH, D), jnp.float32),
            pltpu.VMEM((NSMALL * D,), jnp.float32),
            pltpu.VMEM((NT, 4, CH + 8), jnp.int32),
            pltpu.VMEM((CH + 8,), jnp.int32),
            pltpu.SemaphoreType.DMA,
            pltpu.SemaphoreType.DMA,
            pltpu.SemaphoreType.DMA,
        ],
    )
    return fn(tests, questions, tags, ttypes, qnums, w_test, w_quest,
              w_small)


def kernel(tests, questions, tags, test_types, question_num,
           W_test, W_question, W_tag, W_test_type, W_pos, W_qnum):
    flat = lambda x: x.reshape(-1).astype(jnp.int32)
    w_small = jnp.concatenate(
        [W_pos, W_tag, W_test_type, W_qnum], axis=0).reshape(-1)
    out = _run(flat(tests), flat(questions), flat(tags), flat(test_types),
               flat(question_num), W_test, W_question, w_small)
    return out.reshape(B, S, D)


# speedup vs baseline: 1.0687x; 1.0585x over previous
import functools

import jax
import jax.numpy as jnp
from jax import lax
from jax.experimental import pallas as pl
from jax.experimental.pallas import tpu as pltpu
from jax.experimental.pallas import tpu_sc as plsc

B, S, D = 4096, 200, 64
NW = 32
TOK_PER_W = B * S // NW
L = 16
CH = 40
NCH = TOK_PER_W // CH
NST = 2
NT = 5


def _body(tests, questions, tags, ttypes, qnums,
          w_test, w_quest, w_tag, w_ttype, w_pos, w_qnum,
          out, bufs, outbuf, posv, tagv, ttv, qnv, idxb,
          sem_i, sem_g, sem_o):
    wid = lax.axis_index("s") * 2 + lax.axis_index("c")
    tok0 = wid * TOK_PER_W
    last = NCH - 1

    pltpu.sync_copy(w_pos, posv)
    pltpu.sync_copy(w_tag, tagv)
    pltpu.sync_copy(w_ttype, ttv)
    pltpu.sync_copy(w_qnum, qnv)

    idx_hbms = (questions, tests, tags, ttypes, qnums)
    w_hbms = (w_quest, w_test)

    def fire_idx(c, slot):
        base = tok0 + c * CH
        for i in range(NT):
            pltpu.async_copy(idx_hbms[i].at[pl.ds(base, CH)],
                             idxb.at[i, slot], sem_i)

    def wait_idx():
        for i in range(NT):
            pltpu.make_async_copy(idx_hbms[i].at[pl.ds(0, CH)],
                                  idxb.at[i, 0], sem_i).wait()

    def fire_gather(slot, par):
        for i in range(NST):
            pltpu.async_copy(w_hbms[i].at[idxb.at[i, slot]],
                             bufs.at[i, par], sem_g)

    def wait_gather():
        for i in range(NST):
            pltpu.make_async_copy(w_hbms[i].at[idxb.at[i, 0]],
                                  bufs.at[i, 0], sem_g).wait()

    def wait_out():
        pltpu.make_async_copy(outbuf.at[0], out.at[pl.ds(0, CH)],
                              sem_o).wait()

    fire_idx(0, 0)
    fire_idx(1, 1)
    wait_idx()
    fire_gather(0, 0)

    lanes = lax.iota(jnp.int32, L)

    def step(cc, carry):
        for j in (0, 1, 2, 3):
            par = j % 2
            c = cc * 4 + j
            wait_gather()
            wait_idx()
            fire_gather((j + 1) % 4, 1 - par)
            fire_idx(jnp.minimum(c + 2, last), (j + 2) % 4)

            @pl.when(c >= 2)
            def _():
                wait_out()

            s0 = lax.rem(c, S // CH) * CH

            def tok_body(t, carry2):
                tsplat = jnp.broadcast_to(t, (L,))
                tag_a = plsc.load_gather(idxb.at[2, j], [tsplat]) * D
                tt_a = plsc.load_gather(idxb.at[3, j], [tsplat]) * D
                qn_a = plsc.load_gather(idxb.at[4, j], [tsplat]) * D
                for k in range(D // L):
                    ds = pl.ds(k * L, L)
                    col = lanes + (k * L)
                    v = posv[s0 + t, ds]
                    v = v + bufs[0, par, t, ds]
                    v = v + bufs[1, par, t, ds]
                    v = v + plsc.load_gather(tagv, [tag_a + col])
                    v = v + plsc.load_gather(ttv, [tt_a + col])
                    v = v + plsc.load_gather(qnv, [qn_a + col])
                    outbuf[par, t, ds] = v
                return carry2

            lax.fori_loop(0, CH, tok_body, 0, unroll=4)

            pltpu.async_copy(outbuf.at[par],
                             out.at[pl.ds(tok0 + c * CH, CH)], sem_o)
        return carry

    lax.fori_loop(0, NCH // 4, step, 0)

    wait_out()
    wait_out()
    wait_gather()
    wait_idx()


@jax.jit
def _run(tests, questions, tags, ttypes, qnums,
         w_test, w_quest, w_tag, w_ttype, w_pos, w_qnum):
    mesh = plsc.VectorSubcoreMesh(core_axis_name="c", subcore_axis_name="s")
    fn = pl.kernel(
        _body,
        out_type=jax.ShapeDtypeStruct((B * S, D), jnp.float32),
        mesh=mesh,
        compiler_params=pltpu.CompilerParams(use_tc_tiling_on_sc=False,
                                             needs_layout_passes=False),
        scratch_types=[
            pltpu.VMEM((NST, 2, CH, D), jnp.float32),
            pltpu.VMEM((2, CH, D), jnp.float32),
            pltpu.VMEM((S, D), jnp.float32),
            pltpu.VMEM((1000 * D,), jnp.float32),
            pltpu.VMEM((16 * D,), jnp.float32),
            pltpu.VMEM((64 * D,), jnp.float32),
            pltpu.VMEM((NT, 4, CH), jnp.int32),
            pltpu.SemaphoreType.DMA,
            pltpu.SemaphoreType.DMA,
            pltpu.SemaphoreType.DMA,
        ],
    )
    return fn(tests, questions, tags, ttypes, qnums,
              w_test, w_quest, w_tag.reshape(-1), w_ttype.reshape(-1),
              w_pos, w_qnum.reshape(-1))


def kernel(tests, questions, tags, test_types, question_num,
           W_test, W_question, W_tag, W_test_type, W_pos, W_qnum):
    flat = lambda x: x.reshape(-1).astype(jnp.int32)
    out = _run(flat(tests), flat(questions), flat(tags), flat(test_types),
               flat(question_num),
               W_test, W_question, W_tag, W_test_type, W_pos, W_qnum)
    return out.reshape(B, S, D)
